# baseline (speedup 1.0000x reference)
import jax
import jax.numpy as jnp
from jax import lax
from jax.experimental import pallas as pl
from jax.experimental.pallas import tpu as pltpu
from jax.experimental.pallas import tpu_sc as plsc
import functools

N_NODES = 100000
D_FEAT = 128
NUM_GRAPHS = 1024
OUT_FEATS = 10

NC = 2
NS = 16
NW = NC * NS
ROWS_PER_W = N_NODES // NW
PIECE = 125
IDROW = 128
PIECES = ROWS_PER_W // PIECE
CHUNKS = PIECES // 2
CHUNK = 2 * PIECE
CHUNK_PAD = 2 * IDROW
ACC_ROWS = NUM_GRAPHS + 16
ROWS_PER_TILE = NUM_GRAPHS // NS


def _sc_segment_sums(x, ids2d):
    mesh = plsc.VectorSubcoreMesh(
        core_axis_name="c", subcore_axis_name="s", num_cores=NC, num_subcores=NS
    )

    @functools.partial(
        pl.kernel,
        out_type=(
            jax.ShapeDtypeStruct((NC, NUM_GRAPHS, D_FEAT), jnp.float32),
            jax.ShapeDtypeStruct((NC, NUM_GRAPHS, 16), jnp.float32),
        ),
        mesh=mesh,
        scratch_types=[
            pltpu.VMEM((2, CHUNK_PAD), jnp.int32),
            pltpu.VMEM((2, CHUNK_PAD, D_FEAT), jnp.float32),
            pltpu.VMEM((CHUNK_PAD, 16), jnp.float32),
            pltpu.VMEM((ROWS_PER_TILE, D_FEAT), jnp.float32),
            pltpu.VMEM((ROWS_PER_TILE, 16), jnp.float32),
            pltpu.VMEM_SHARED((ACC_ROWS, D_FEAT), jnp.float32),
            pltpu.VMEM_SHARED((ACC_ROWS, 16), jnp.float32),
            pltpu.SemaphoreType.DMA,
            pltpu.SemaphoreType.DMA,
            pltpu.SemaphoreType.DMA,
            pltpu.SemaphoreType.DMA,
        ],
        compiler_params=pltpu.CompilerParams(use_tc_tiling_on_sc=False),
    )
    def k(x_hbm, ids_hbm, psums, pcnts, idx_v, rows_v, ones_v, zsum_v, zcnt_v,
          acc_sh, cnt_sh, si0, si1, sr0, sr1):
        c = lax.axis_index("c")
        s = lax.axis_index("s")
        wid = c * NS + s
        row0 = wid * ROWS_PER_W
        q0 = wid * PIECES
        id_sems = (si0, si1)
        row_sems = (sr0, sr1)

        def gather_copies(kk, slot, tail):
            npc = 1 if tail else 2
            cps = []
            for p in range(npc):
                cps.append(
                    pltpu.make_async_copy(
                        ids_hbm.at[q0 + 2 * kk + p],
                        idx_v.at[slot, pl.ds(p * IDROW, IDROW)],
                        id_sems[slot],
                    )
                )
                cps.append(
                    pltpu.make_async_copy(
                        x_hbm.at[pl.ds(row0 + (2 * kk + p) * PIECE, PIECE)],
                        rows_v.at[slot, pl.ds(p * IDROW, PIECE)],
                        row_sems[slot],
                    )
                )
            return cps

        def start_gather(kk, slot, tail=False):
            for cp in gather_copies(kk, slot, tail):
                cp.start()

        def wait_gather(kk, slot, tail=False):
            for cp in gather_copies(kk, slot, tail):
                cp.wait()

        start_gather(0, 0)
        start_gather(1, 1)

        def init_ones(i, carry):
            ones_v[i, :] = jnp.ones((16,), jnp.float32)
            return carry

        lax.fori_loop(0, CHUNK_PAD, init_ones, 0)

        def init_zero(i, carry):
            for j in range(D_FEAT // 16):
                zsum_v[i, pl.ds(j * 16, 16)] = jnp.zeros((16,), jnp.float32)
            zcnt_v[i, :] = jnp.zeros((16,), jnp.float32)
            return carry

        lax.fori_loop(0, ROWS_PER_TILE, init_zero, 0)

        for slot in range(2):
            for p in range(2):
                for r in range(p * IDROW + PIECE, (p + 1) * IDROW):
                    for j in range(D_FEAT // 16):
                        rows_v[slot, r, pl.ds(j * 16, 16)] = jnp.zeros(
                            (16,), jnp.float32
                        )

        pltpu.sync_copy(zsum_v, acc_sh.at[pl.ds(s * ROWS_PER_TILE, ROWS_PER_TILE)])
        pltpu.sync_copy(zcnt_v, cnt_sh.at[pl.ds(s * ROWS_PER_TILE, ROWS_PER_TILE)])
        plsc.subcore_barrier()

        for kk in range(CHUNKS + 1):
            slot = kk % 2
            tail = kk == CHUNKS
            wait_gather(kk, slot, tail)
            if tail:
                dummy = jnp.full((16,), NUM_GRAPHS, jnp.int32)
                for t in range(IDROW // 16):
                    idx_v[slot, pl.ds(IDROW + 16 * t, 16)] = dummy
            pltpu.sync_copy(
                rows_v.at[slot], acc_sh.at[idx_v.at[slot]], add=True
            )
            pltpu.sync_copy(ones_v, cnt_sh.at[idx_v.at[slot]], add=True)
            if kk + 2 <= CHUNKS:
                start_gather(kk + 2, slot, tail=(kk + 2 == CHUNKS))
        plsc.subcore_barrier()

        sl = pl.ds(s * ROWS_PER_TILE, ROWS_PER_TILE)
        pltpu.sync_copy(acc_sh.at[sl], psums.at[c, sl])
        pltpu.sync_copy(cnt_sh.at[sl], pcnts.at[c, sl])

    return k(x, ids2d)


def _tc_finish(psums, pcnts, W, b2):
    def body(ps_hbm, pc_hbm, w, b, out, ps, pc, sem1, sem2):
        cp1 = pltpu.make_async_copy(ps_hbm, ps, sem1)
        cp2 = pltpu.make_async_copy(pc_hbm, pc, sem2)
        cp1.start()
        cp2.start()
        cp1.wait()
        cp2.wait()
        sums = ps[0] + ps[1]
        cnt = (pc[0] + pc[1])[:, 0:1]
        mean = sums / jnp.maximum(cnt, 1.0)
        out[...] = (
            lax.dot_general(
                mean, w[...], (((1,), (1,)), ((), ())),
                preferred_element_type=jnp.float32,
            )
            + b[...]
        )

    return pl.pallas_call(
        body,
        in_specs=[
            pl.BlockSpec(memory_space=pltpu.HBM),
            pl.BlockSpec(memory_space=pltpu.HBM),
            pl.BlockSpec(memory_space=pltpu.VMEM),
            pl.BlockSpec(memory_space=pltpu.VMEM),
        ],
        scratch_shapes=[
            pltpu.VMEM((NC, NUM_GRAPHS, D_FEAT), jnp.float32),
            pltpu.VMEM((NC, NUM_GRAPHS, 16), jnp.float32),
            pltpu.SemaphoreType.DMA,
            pltpu.SemaphoreType.DMA,
        ],
        out_shape=jax.ShapeDtypeStruct((NUM_GRAPHS, OUT_FEATS), jnp.float32),
    )(psums, pcnts, W, b2)


def kernel(inputs, segment_ids, W, b):
    ids32 = segment_ids.astype(jnp.int32).reshape(NW * PIECES, PIECE)
    ids2d = jnp.pad(
        ids32, ((0, 0), (0, IDROW - PIECE)), constant_values=NUM_GRAPHS
    )
    psums, pcnts = _sc_segment_sums(inputs, ids2d)
    return _tc_finish(psums, pcnts, W, b.reshape(1, OUT_FEATS))


# speedup vs baseline: 1.1572x; 1.1572x over previous
import jax
import jax.numpy as jnp
from jax import lax
from jax.experimental import pallas as pl
from jax.experimental.pallas import tpu as pltpu
from jax.experimental.pallas import tpu_sc as plsc
import functools

N_NODES = 100000
D_FEAT = 128
NUM_GRAPHS = 1024
OUT_FEATS = 10

NC = 2
NS = 16
NW = NC * NS
CHUNK = 125
CHUNK_PAD = 128
TOT_CHUNKS = N_NODES // CHUNK
CHUNKS_C0 = 23
CHUNKS_C1 = 27
CHUNKS_MAX = max(CHUNKS_C0, CHUNKS_C1)
ACC_ROWS = NUM_GRAPHS + 16
ROWS_PER_TILE = NUM_GRAPHS // NS


def _sc_segment_sums(x, ids2d):
    mesh = plsc.VectorSubcoreMesh(
        core_axis_name="c", subcore_axis_name="s", num_cores=NC, num_subcores=NS
    )

    @functools.partial(
        pl.kernel,
        out_type=(
            jax.ShapeDtypeStruct((NC, NUM_GRAPHS, D_FEAT), jnp.float32),
            jax.ShapeDtypeStruct((NC, NUM_GRAPHS, 16), jnp.float32),
        ),
        mesh=mesh,
        scratch_types=[
            pltpu.VMEM((2, CHUNK_PAD), jnp.int32),
            pltpu.VMEM((2, CHUNK_PAD, D_FEAT), jnp.float32),
            pltpu.VMEM((CHUNK_PAD, 16), jnp.float32),
            pltpu.VMEM((ROWS_PER_TILE, D_FEAT), jnp.float32),
            pltpu.VMEM((ROWS_PER_TILE, 16), jnp.float32),
            pltpu.VMEM_SHARED((ACC_ROWS, D_FEAT), jnp.float32),
            pltpu.VMEM_SHARED((ACC_ROWS, 16), jnp.float32),
            pltpu.SemaphoreType.DMA,
            pltpu.SemaphoreType.DMA,
            pltpu.SemaphoreType.DMA,
            pltpu.SemaphoreType.DMA,
        ],
        compiler_params=pltpu.CompilerParams(use_tc_tiling_on_sc=False),
    )
    def k(x_hbm, ids_hbm, psums, pcnts, idx_v, rows_v, ones_v, zsum_v, zcnt_v,
          acc_sh, cnt_sh, si0, si1, sr0, sr1):
        c = lax.axis_index("c")
        s = lax.axis_index("s")
        my_chunks = jnp.where(c == 0, CHUNKS_C0, CHUNKS_C1)
        q0 = jnp.where(
            c == 0, s * CHUNKS_C0, NS * CHUNKS_C0 + s * CHUNKS_C1
        )
        row0 = q0 * CHUNK
        id_sems = (si0, si1)
        row_sems = (sr0, sr1)

        def gather_copies(kk, slot):
            return (
                pltpu.make_async_copy(
                    ids_hbm.at[q0 + kk], idx_v.at[slot], id_sems[slot]
                ),
                pltpu.make_async_copy(
                    x_hbm.at[pl.ds(row0 + kk * CHUNK, CHUNK)],
                    rows_v.at[slot, pl.ds(0, CHUNK)],
                    row_sems[slot],
                ),
            )

        def start_gather(kk, slot):
            for cp in gather_copies(kk, slot):
                cp.start()

        def wait_gather(kk, slot):
            for cp in gather_copies(kk, slot):
                cp.wait()

        start_gather(0, 0)
        start_gather(1, 1)

        def init_ones(i, carry):
            ones_v[i, :] = jnp.ones((16,), jnp.float32)
            return carry

        lax.fori_loop(0, CHUNK_PAD, init_ones, 0)

        def init_zero(i, carry):
            for j in range(D_FEAT // 16):
                zsum_v[i, pl.ds(j * 16, 16)] = jnp.zeros((16,), jnp.float32)
            zcnt_v[i, :] = jnp.zeros((16,), jnp.float32)
            return carry

        lax.fori_loop(0, ROWS_PER_TILE, init_zero, 0)

        for slot in range(2):
            for r in range(CHUNK, CHUNK_PAD):
                for j in range(D_FEAT // 16):
                    rows_v[slot, r, pl.ds(j * 16, 16)] = jnp.zeros(
                        (16,), jnp.float32
                    )

        pltpu.sync_copy(zsum_v, acc_sh.at[pl.ds(s * ROWS_PER_TILE, ROWS_PER_TILE)])
        pltpu.sync_copy(zcnt_v, cnt_sh.at[pl.ds(s * ROWS_PER_TILE, ROWS_PER_TILE)])
        plsc.subcore_barrier()

        for kk in range(CHUNKS_MAX):
            slot = kk % 2

            @pl.when(kk < my_chunks)
            def _():
                wait_gather(kk, slot)
                pltpu.sync_copy(
                    rows_v.at[slot], acc_sh.at[idx_v.at[slot]], add=True
                )
                pltpu.sync_copy(ones_v, cnt_sh.at[idx_v.at[slot]], add=True)

                @pl.when(kk + 2 < my_chunks)
                def _():
                    start_gather(kk + 2, slot)

        plsc.subcore_barrier()

        sl = pl.ds(s * ROWS_PER_TILE, ROWS_PER_TILE)
        pltpu.sync_copy(acc_sh.at[sl], psums.at[c, sl])
        pltpu.sync_copy(cnt_sh.at[sl], pcnts.at[c, sl])

    return k(x, ids2d)


def _tc_finish(psums, pcnts, W, b2):
    def body(ps, pc, w, b, out):
        sums = ps[0] + ps[1]
        cnt = (pc[0] + pc[1])[:, 0:1]
        mean = sums / jnp.maximum(cnt, 1.0)
        out[...] = (
            lax.dot_general(
                mean, w[...], (((1,), (1,)), ((), ())),
                preferred_element_type=jnp.float32,
            )
            + b[...]
        )

    return pl.pallas_call(
        body,
        out_shape=jax.ShapeDtypeStruct((NUM_GRAPHS, OUT_FEATS), jnp.float32),
    )(psums, pcnts, W, b2)


def kernel(inputs, segment_ids, W, b):
    ids32 = segment_ids.astype(jnp.int32).reshape(TOT_CHUNKS, CHUNK)
    ids2d = jnp.pad(
        ids32, ((0, 0), (0, CHUNK_PAD - CHUNK)), constant_values=NUM_GRAPHS
    )
    psums, pcnts = _sc_segment_sums(inputs, ids2d)
    return _tc_finish(psums, pcnts, W, b.reshape(1, OUT_FEATS))
